# baseline (speedup 1.0000x reference)
import functools

import jax
import jax.numpy as jnp
from jax import lax
from jax.experimental import pallas as pl
from jax.experimental.pallas import tpu as pltpu
from jax.experimental.pallas import tpu_sc as plsc

B, T, D = 4, 2048, 1024
L = 16
NC, NS = 2, 16
NW = NC * NS
T_PER_W = T // NW
CT = 8
NSTEP = T_PER_W // CT
ROWS = B * CT
VECS = D // L
NBUF = 3
UNROLL = 4

_mesh = plsc.VectorSubcoreMesh(core_axis_name="c", subcore_axis_name="s")


@functools.partial(
    pl.kernel,
    mesh=_mesh,
    out_type=jax.ShapeDtypeStruct((B, T, D), jnp.float32),
    scratch_types=[
        pltpu.VMEM((B * T_PER_W,), jnp.int32),
        pltpu.VMEM((NBUF, CT, D), jnp.float32),
        pltpu.VMEM((NBUF, ROWS, D), jnp.float32),
    ] + [pltpu.SemaphoreType.DMA] * (3 * NBUF + 1),
)
def _embed(idx_hbm, wpe_hbm, wte_hbm, out_hbm, idx2_v, wpe_v, rows_v,
           *sems):
    gsem = sems[0:NBUF]
    wsem = sems[NBUF:2 * NBUF]
    osem = sems[2 * NBUF:3 * NBUF]
    isem = sems[3 * NBUF]
    wid = lax.axis_index("s") * NC + lax.axis_index("c")
    t_base = wid * T_PER_W

    idx_handles = [
        pltpu.async_copy(idx_hbm.at[b, pl.ds(t_base + c * CT, CT)],
                         idx2_v.at[pl.ds(c * ROWS + b * CT, CT)], isem)
        for c in range(NSTEP)
        for b in range(B)
    ]
    for h in idx_handles:
        h.wait()

    def start_step(c):
        buf = c % NBUF
        t0 = t_base + c * CT
        iv = idx2_v.at[pl.ds(c * ROWS, ROWS)]
        return [
            pltpu.async_copy(wte_hbm.at[iv], rows_v.at[buf], gsem[buf]),
            pltpu.async_copy(wpe_hbm.at[pl.ds(t0, CT)], wpe_v.at[buf],
                             wsem[buf]),
        ]

    def compute_step(c):
        buf = c % NBUF

        def v_body(i, _):
            tl = i // (VECS // UNROLL)
            colbase = (i % (VECS // UNROLL)) * (UNROLL * L)
            for u in range(UNROLL):
                col = colbase + u * L
                w = wpe_v[buf, tl, pl.ds(col, L)]
                for b in range(B):
                    plsc.addupdate(rows_v.at[buf, b * CT + tl, pl.ds(col, L)],
                                   w)
            return 0

        lax.fori_loop(0, CT * VECS // UNROLL, v_body, 0)

    def start_out(c):
        buf = c % NBUF
        t0 = t_base + c * CT
        return [
            pltpu.async_copy(rows_v.at[buf, pl.ds(b * CT, CT)],
                             out_hbm.at[b, pl.ds(t0, CT)], osem[buf])
            for b in range(B)
        ]

    pending = {0: start_step(0), 1: start_step(1)}
    out_handles = {}
    for c in range(NSTEP):
        for h in pending.pop(c):
            h.wait()
        compute_step(c)
        out_handles[c] = start_out(c)
        if c + 2 < NSTEP:
            if c - 1 >= 0:
                for h in out_handles.pop(c - 1):
                    h.wait()
            pending[c + 2] = start_step(c + 2)
    for c in out_handles:
        for h in out_handles[c]:
            h.wait()


def kernel(idx, wpe_table, wte_table):
    return _embed(idx.astype(jnp.int32), wpe_table, wte_table)


# speedup vs baseline: 1.0061x; 1.0061x over previous
import functools

import jax
import jax.numpy as jnp
from jax import lax
from jax.experimental import pallas as pl
from jax.experimental.pallas import tpu as pltpu
from jax.experimental.pallas import tpu_sc as plsc

B, T, D = 4, 2048, 1024
L = 16
NC, NS = 2, 16
NW = NC * NS
T_PER_W = T // NW
CT = 8
NSTEP = T_PER_W // CT
ROWS = B * CT
VECS = D // L
NBUF = 3
UNROLL = 8

_mesh = plsc.VectorSubcoreMesh(core_axis_name="c", subcore_axis_name="s")


@functools.partial(
    pl.kernel,
    mesh=_mesh,
    out_type=jax.ShapeDtypeStruct((B, T, D), jnp.float32),
    scratch_types=[
        pltpu.VMEM((B * T_PER_W,), jnp.int32),
        pltpu.VMEM((NBUF, CT, D), jnp.float32),
        pltpu.VMEM((NBUF, ROWS, D), jnp.float32),
    ] + [pltpu.SemaphoreType.DMA] * (3 * NBUF + 1),
)
def _embed(idx_hbm, wpe_hbm, wte_hbm, out_hbm, idx2_v, wpe_v, rows_v,
           *sems):
    gsem = sems[0:NBUF]
    wsem = sems[NBUF:2 * NBUF]
    osem = sems[2 * NBUF:3 * NBUF]
    isem = sems[3 * NBUF]
    wid = lax.axis_index("s") * NC + lax.axis_index("c")
    t_base = wid * T_PER_W

    idx_handles = [
        pltpu.async_copy(idx_hbm.at[b, pl.ds(t_base + c * CT, CT)],
                         idx2_v.at[pl.ds(c * ROWS + b * CT, CT)], isem)
        for c in range(NSTEP)
        for b in range(B)
    ]
    for h in idx_handles:
        h.wait()

    def start_step(c):
        buf = c % NBUF
        t0 = t_base + c * CT
        iv = idx2_v.at[pl.ds(c * ROWS, ROWS)]
        return [
            pltpu.async_copy(wte_hbm.at[iv], rows_v.at[buf], gsem[buf]),
            pltpu.async_copy(wpe_hbm.at[pl.ds(t0, CT)], wpe_v.at[buf],
                             wsem[buf]),
        ]

    def compute_step(c):
        buf = c % NBUF

        def v_body(i, _):
            tl = i // (VECS // UNROLL)
            colbase = (i % (VECS // UNROLL)) * (UNROLL * L)
            for u in range(UNROLL):
                col = colbase + u * L
                w = wpe_v[buf, tl, pl.ds(col, L)]
                for b in range(B):
                    plsc.addupdate(rows_v.at[buf, b * CT + tl, pl.ds(col, L)],
                                   w)
            return 0

        lax.fori_loop(0, CT * VECS // UNROLL, v_body, 0)

    def start_out(c):
        buf = c % NBUF
        t0 = t_base + c * CT
        return [
            pltpu.async_copy(rows_v.at[buf, pl.ds(b * CT, CT)],
                             out_hbm.at[b, pl.ds(t0, CT)], osem[buf])
            for b in range(B)
        ]

    pending = {0: start_step(0), 1: start_step(1)}
    out_handles = {}
    for c in range(NSTEP):
        for h in pending.pop(c):
            h.wait()
        compute_step(c)
        out_handles[c] = start_out(c)
        if c + 2 < NSTEP:
            if c - 1 >= 0:
                for h in out_handles.pop(c - 1):
                    h.wait()
            pending[c + 2] = start_step(c + 2)
    for c in out_handles:
        for h in out_handles[c]:
            h.wait()


def kernel(idx, wpe_table, wte_table):
    return _embed(idx.astype(jnp.int32), wpe_table, wte_table)


# speedup vs baseline: 1.0061x; 1.0000x over previous
import functools

import jax
import jax.numpy as jnp
from jax import lax
from jax.experimental import pallas as pl
from jax.experimental.pallas import tpu as pltpu
from jax.experimental.pallas import tpu_sc as plsc

B, T, D = 4, 2048, 1024
L = 16
NC, NS = 2, 16
NW = NC * NS
T_PER_W = T // NW
CT = 8
NSTEP = T_PER_W // CT
ROWS = B * CT
VECS = D // L
NBUF = 3
UNROLL = 8

_mesh = plsc.VectorSubcoreMesh(core_axis_name="c", subcore_axis_name="s")


@functools.partial(
    pl.kernel,
    mesh=_mesh,
    out_type=jax.ShapeDtypeStruct((B, T, D), jnp.float32),
    scratch_types=[
        pltpu.VMEM((NSTEP * ROWS,), jnp.int32),
        pltpu.VMEM((NBUF, CT, D), jnp.float32),
        pltpu.VMEM((NBUF, ROWS, D), jnp.float32),
    ] + [pltpu.SemaphoreType.DMA] * (3 * NBUF + 1),
)
def _embed(idx_hbm, wpe_hbm, wte_hbm, out_hbm, idx2_v, wpe_v, rows_v, *sems):
    gsem = sems[0:NBUF]
    wsem = sems[NBUF:2 * NBUF]
    osem = sems[2 * NBUF:3 * NBUF]
    isem = sems[3 * NBUF]
    wid = lax.axis_index("s") * NC + lax.axis_index("c")
    t_base = wid * T_PER_W

    idx_handles = [
        pltpu.async_copy(idx_hbm.at[b, pl.ds(t_base + c * CT, CT)],
                         idx2_v.at[pl.ds(c * ROWS + b * CT, CT)], isem)
        for c in range(NSTEP)
        for b in range(B)
    ]
    for h in idx_handles:
        h.wait()

    def start_step(c):
        buf = c % NBUF
        t0 = t_base + c * CT
        iv = idx2_v.at[pl.ds(c * ROWS, ROWS)]
        return [
            pltpu.async_copy(wte_hbm.at[iv], rows_v.at[buf], gsem[buf]),
            pltpu.async_copy(wpe_hbm.at[pl.ds(t0, CT)], wpe_v.at[buf],
                             wsem[buf]),
        ]

    def compute_step(c):
        buf = c % NBUF

        def v_body(i, _):
            tl = i // (VECS // UNROLL)
            colbase = (i % (VECS // UNROLL)) * (UNROLL * L)
            for u in range(UNROLL):
                col = colbase + u * L
                w = wpe_v[buf, tl, pl.ds(col, L)]
                for b in range(B):
                    plsc.addupdate(rows_v.at[buf, b * CT + tl, pl.ds(col, L)],
                                   w)
            return 0

        lax.fori_loop(0, CT * VECS // UNROLL, v_body, 0)

    def start_out(c):
        buf = c % NBUF
        t0 = t_base + c * CT
        return [
            pltpu.async_copy(rows_v.at[buf, pl.ds(b * CT, CT)],
                             out_hbm.at[b, pl.ds(t0, CT)], osem[buf])
            for b in range(B)
        ]

    pending = {0: start_step(0), 1: start_step(1)}
    out_handles = {}
    for c in range(NSTEP):
        for h in pending.pop(c):
            h.wait()
        compute_step(c)
        out_handles[c] = start_out(c)
        if c + 2 < NSTEP:
            if c - 1 >= 0:
                for h in out_handles.pop(c - 1):
                    h.wait()
            pending[c + 2] = start_step(c + 2)
    for c in out_handles:
        for h in out_handles[c]:
            h.wait()


def kernel(idx, wpe_table, wte_table):
    return _embed(idx.astype(jnp.int32), wpe_table, wte_table)


# speedup vs baseline: 1.0351x; 1.0288x over previous
import functools

import jax
import jax.numpy as jnp
from jax import lax
from jax.experimental import pallas as pl
from jax.experimental.pallas import tpu as pltpu
from jax.experimental.pallas import tpu_sc as plsc

B, T, D = 4, 2048, 1024
L = 16
NC, NS = 2, 16
NW = NC * NS
T_PER_W = T // NW
CT = 8
NSTEP = T_PER_W // CT
ROWS = B * CT
VECS = D // L
NBUF = 3
UNROLL = 8

_mesh = plsc.VectorSubcoreMesh(core_axis_name="c", subcore_axis_name="s")


@functools.partial(
    pl.kernel,
    mesh=_mesh,
    out_type=jax.ShapeDtypeStruct((B, T, D), jnp.float32),
    scratch_types=[
        pltpu.VMEM((NSTEP * ROWS,), jnp.int32),
        pltpu.VMEM((NBUF, CT, D), jnp.float32),
        pltpu.VMEM((NBUF, ROWS, D), jnp.float32),
    ] + [pltpu.SemaphoreType.DMA] * (3 * NBUF + 1),
)
def _embed(idx_hbm, wpe_hbm, wte_hbm, out_hbm, idx2_v, wpe_v, rows_v, *sems):
    gsem = sems[0:NBUF]
    wsem = sems[NBUF:2 * NBUF]
    osem = sems[2 * NBUF:3 * NBUF]
    isem = sems[3 * NBUF]
    wid = lax.axis_index("s") * NC + lax.axis_index("c")
    t_base = wid * T_PER_W

    idx_handles = [
        pltpu.async_copy(idx_hbm.at[b, pl.ds(t_base + c * CT, CT)],
                         idx2_v.at[pl.ds(c * ROWS + b * CT, CT)], isem)
        for c in range(NSTEP)
        for b in range(B)
    ]
    for h in idx_handles:
        h.wait()

    def start_step(c, buf):
        t0 = t_base + c * CT
        iv = idx2_v.at[pl.ds(c * ROWS, ROWS)]
        return [
            pltpu.async_copy(wte_hbm.at[iv], rows_v.at[buf], gsem[buf]),
            pltpu.async_copy(wpe_hbm.at[pl.ds(t0, CT)], wpe_v.at[buf],
                             wsem[buf]),
        ]

    def compute_step(c, buf):
        def v_body(i, _):
            tl = i // (VECS // UNROLL)
            colbase = (i % (VECS // UNROLL)) * (UNROLL * L)
            for u in range(UNROLL):
                col = colbase + u * L
                w = wpe_v[buf, tl, pl.ds(col, L)]
                for b in range(B):
                    plsc.addupdate(rows_v.at[buf, b * CT + tl, pl.ds(col, L)],
                                   w)
            return 0

        lax.fori_loop(0, CT * VECS // UNROLL, v_body, 0)

    def start_out(c, buf):
        t0 = t_base + c * CT
        return [
            pltpu.async_copy(rows_v.at[buf, pl.ds(b * CT, CT)],
                             out_hbm.at[b, pl.ds(t0, CT)], osem[buf])
            for b in range(B)
        ]

    def wait_step(c, buf):
        pltpu.make_async_copy(out_hbm.at[0, pl.ds(0, ROWS)], rows_v.at[buf],
                              gsem[buf]).wait()
        pltpu.make_async_copy(wpe_hbm.at[pl.ds(t_base + c * CT, CT)],
                              wpe_v.at[buf], wsem[buf]).wait()

    def wait_out(c, buf):
        t0 = t_base + c * CT
        for b in range(B):
            pltpu.make_async_copy(rows_v.at[buf, pl.ds(b * CT, CT)],
                                  out_hbm.at[b, pl.ds(t0, CT)],
                                  osem[buf]).wait()

    start_step(0, 0)
    start_step(1, 1)

    def group_body(g, _):
        for j in range(NBUF):
            c = g * NBUF + j
            wait_step(c, j)
            compute_step(c, j)
            start_out(c, j)

            @pl.when(c >= 1)
            def _():
                wait_out(c - 1, (j + 2) % NBUF)
            start_step(c + 2, (j + 2) % NBUF)
        return 0

    lax.fori_loop(0, (NSTEP - 2) // NBUF, group_body, 0)

    for c in range(NSTEP - 2, NSTEP):
        wait_step(c, c % NBUF)
        compute_step(c, c % NBUF)
        start_out(c, c % NBUF)
    for c in range(NSTEP - 3, NSTEP):
        wait_out(c, c % NBUF)


def kernel(idx, wpe_table, wte_table):
    return _embed(idx.astype(jnp.int32), wpe_table, wte_table)
